# baseline (speedup 1.0000x reference)
import functools

import jax
import jax.numpy as jnp
import numpy as np
from jax import lax
from jax.experimental import pallas as pl
from jax.experimental.pallas import tpu as pltpu
from jax.experimental.pallas import tpu_sc as plsc

_L = 8
_T = 1024
_B_G = float(np.exp(np.log(20 * 0.5 / 2) / (_L - 1)))
_RES = [int(np.floor(2 * _B_G**l)) for l in range(_L)]
_C1 = np.int32(np.uint32(2654435761).astype(np.int32))
_C2 = np.int32(805459861)

_NW = 32
_CNK = 4096
_C1_2 = np.int32(np.uint32(2 * 2654435761 % (2**32)).astype(np.int32))
_C2_2 = np.int32(np.uint32(2 * 805459861 % (2**32)).astype(np.int32))


def _sc_encode(xs_a, ys_a, zs_a, grid_flat):
    n = xs_a.shape[0]
    npts = n // _NW
    nchunk = npts // _CNK
    mesh = plsc.VectorSubcoreMesh(core_axis_name="c", subcore_axis_name="s")

    @functools.partial(
        pl.kernel,
        out_type=jax.ShapeDtypeStruct((16, n), jnp.float32),
        mesh=mesh,
        scratch_types=[
            pltpu.VMEM((_L * _T * 2,), jnp.float32),
            pltpu.VMEM((_CNK,), jnp.float32),
            pltpu.VMEM((_CNK,), jnp.float32),
            pltpu.VMEM((_CNK,), jnp.float32),
            pltpu.VMEM((16, _CNK), jnp.float32),
        ],
        compiler_params=pltpu.CompilerParams(needs_layout_passes=False),
    )
    def enc(xs_hbm, ys_hbm, zs_hbm, grid_hbm, ft_hbm, gv, xv, yv, zv, fv):
        wid = lax.axis_index("s") * 2 + lax.axis_index("c")
        pltpu.sync_copy(grid_hbm, gv)
        base0 = wid * npts

        def chunk_body(ci, _):
            base = base0 + ci * _CNK
            pltpu.sync_copy(xs_hbm.at[pl.ds(base, _CNK)], xv)
            pltpu.sync_copy(ys_hbm.at[pl.ds(base, _CNK)], yv)
            pltpu.sync_copy(zs_hbm.at[pl.ds(base, _CNK)], zv)

            def pt_body(i, _):
                sl = pl.ds(i * 16, 16)
                xs = xv[sl]
                ys = yv[sl]
                zs = zv[sl]
                for l in range(_L):
                    res = float(_RES[l])
                    loff = l * _T * 2
                    px = xs * res
                    py = ys * res
                    pz = zs * res
                    ix = px.astype(jnp.int32)
                    iy = py.astype(jnp.int32)
                    iz = pz.astype(jnp.int32)
                    wx1 = px - ix.astype(jnp.float32)
                    wy1 = py - iy.astype(jnp.float32)
                    wz1 = pz - iz.astype(jnp.float32)
                    ix2 = ix * 2
                    hy20 = iy * _C1_2
                    hz20 = iz * _C2_2
                    hx = (ix2, ix2 + 2)
                    hy = (hy20, hy20 + _C1_2)
                    hz = (hz20, hz20 + _C2_2)
                    wxs = (1.0 - wx1, wx1)
                    wys = (1.0 - wy1, wy1)
                    wzs = (1.0 - wz1, wz1)
                    acc0 = None
                    acc1 = None
                    for dx in (0, 1):
                        for dy in (0, 1):
                            hxy = hx[dx] ^ hy[dy]
                            wxy = wxs[dx] * wys[dy]
                            for dz in (0, 1):
                                idx = ((hxy ^ hz[dz]) & (2 * _T - 1)) | loff
                                g0 = plsc.load_gather(gv, [idx])
                                g1 = plsc.load_gather(gv, [idx + 1])
                                w = wxy * wzs[dz]
                                if acc0 is None:
                                    acc0 = w * g0
                                    acc1 = w * g1
                                else:
                                    acc0 = acc0 + w * g0
                                    acc1 = acc1 + w * g1
                    fv[2 * l, sl] = acc0
                    fv[2 * l + 1, sl] = acc1
                return 0

            lax.fori_loop(0, _CNK // 16, pt_body, 0, unroll=2)
            pltpu.sync_copy(fv, ft_hbm.at[:, pl.ds(base, _CNK)])
            return 0

        lax.fori_loop(0, nchunk, chunk_body, 0)

    return enc(xs_a, ys_a, zs_a, grid_flat)


def _mlp_body(ft_ref, w1_ref, w2_ref, r1_ref, r2_ref, r3_ref, o_ref):
    f32 = jnp.float32
    cn = (((0,), (0,)), ((), ()))
    ft = ft_ref[...]
    t = lax.dot_general(w1_ref[...], ft, cn,
                        preferred_element_type=f32)
    t = jnp.maximum(t, 0.0)
    w21 = jnp.dot(w2_ref[...], r1_ref[...], preferred_element_type=f32)
    r = jnp.maximum(lax.dot_general(w21, t, cn,
                                    preferred_element_type=f32), 0.0)
    r = jnp.maximum(lax.dot_general(r2_ref[...], r, cn,
                                    preferred_element_type=f32), 0.0)
    o_ref[...] = jax.nn.sigmoid(
        lax.dot_general(r3_ref[...], r, cn, preferred_element_type=f32))


@functools.partial(jax.jit, static_argnames=("nb",))
def _run(x, grid, W1, W2, R1, R2, R3, nb=16384):
    n = x.shape[0]
    ft = _sc_encode(x[:, 0], x[:, 1], x[:, 2], grid.reshape(-1))
    full = lambda a: pl.BlockSpec(a.shape, lambda i: (0,) * a.ndim)
    out = pl.pallas_call(
        _mlp_body,
        grid=(n // nb,),
        in_specs=[
            pl.BlockSpec((16, nb), lambda i: (0, i)),
            full(W1), full(W2), full(R1), full(R2), full(R3),
        ],
        out_specs=pl.BlockSpec((1, nb), lambda i: (0, i)),
        out_shape=jax.ShapeDtypeStruct((1, n), jnp.float32),
    )(ft, W1, W2, R1, R2, R3)
    return out.reshape(n, 1)


def kernel(x, grid, W1, W2, R1, R2, R3):
    return _run(x, grid, W1, W2, R1, R2, R3)


# speedup vs baseline: 1.0870x; 1.0870x over previous
import functools

import jax
import jax.numpy as jnp
import numpy as np
from jax import lax
from jax.experimental import pallas as pl
from jax.experimental.pallas import tpu as pltpu
from jax.experimental.pallas import tpu_sc as plsc

_L = 8
_T = 1024
_B_G = float(np.exp(np.log(20 * 0.5 / 2) / (_L - 1)))
_RES = [int(np.floor(2 * _B_G**l)) for l in range(_L)]
_C1 = np.int32(np.uint32(2654435761).astype(np.int32))
_C2 = np.int32(805459861)

_NW = 32
_CNK = 4096
_C1_2 = np.int32(np.uint32(2 * 2654435761 % (2**32)).astype(np.int32))
_C2_2 = np.int32(np.uint32(2 * 805459861 % (2**32)).astype(np.int32))


def _sc_encode(xs_a, ys_a, zs_a, grid_flat):
    n = xs_a.shape[0]
    npts = n // _NW
    nchunk = npts // _CNK
    mesh = plsc.VectorSubcoreMesh(core_axis_name="c", subcore_axis_name="s")

    @functools.partial(
        pl.kernel,
        out_type=jax.ShapeDtypeStruct((16, n), jnp.float32),
        mesh=mesh,
        scratch_types=[
            pltpu.VMEM((_L * _T * 2,), jnp.float32),
            pltpu.VMEM((_CNK,), jnp.float32),
            pltpu.VMEM((_CNK,), jnp.float32),
            pltpu.VMEM((_CNK,), jnp.float32),
            pltpu.VMEM((16, _CNK), jnp.float32),
        ],
        compiler_params=pltpu.CompilerParams(needs_layout_passes=False),
    )
    def enc(xs_hbm, ys_hbm, zs_hbm, grid_hbm, ft_hbm, gv, xv, yv, zv, fv):
        wid = lax.axis_index("s") * 2 + lax.axis_index("c")
        pltpu.sync_copy(grid_hbm, gv)
        base0 = wid * npts

        def chunk_body(ci, _):
            base = base0 + ci * _CNK
            pltpu.sync_copy(xs_hbm.at[pl.ds(base, _CNK)], xv)
            pltpu.sync_copy(ys_hbm.at[pl.ds(base, _CNK)], yv)
            pltpu.sync_copy(zs_hbm.at[pl.ds(base, _CNK)], zv)

            def pt_body(i, _):
                sl = pl.ds(i * 16, 16)
                xs = xv[sl]
                ys = yv[sl]
                zs = zv[sl]
                for l in range(_L):
                    res = float(_RES[l])
                    loff = l * _T * 2
                    px = xs * res
                    py = ys * res
                    pz = zs * res
                    ix = px.astype(jnp.int32)
                    iy = py.astype(jnp.int32)
                    iz = pz.astype(jnp.int32)
                    wx1 = px - ix.astype(jnp.float32)
                    wy1 = py - iy.astype(jnp.float32)
                    wz1 = pz - iz.astype(jnp.float32)
                    hy0 = iy * _C1
                    hz0 = iz * _C2
                    hx = (ix, ix + 1)
                    hy = (hy0, hy0 + _C1)
                    hz = (hz0, hz0 + _C2)
                    wxs = (1.0 - wx1, wx1)
                    wys = (1.0 - wy1, wy1)
                    wzs = (1.0 - wz1, wz1)
                    acc0 = None
                    acc1 = None
                    for dx in (0, 1):
                        for dy in (0, 1):
                            hxy = hx[dx] ^ hy[dy]
                            wxy = wxs[dx] * wys[dy]
                            for dz in (0, 1):
                                idx = ((hxy ^ hz[dz]) & (_T - 1)) | loff
                                g0 = plsc.load_gather(gv, [idx])
                                g1 = plsc.load_gather(gv, [idx + _T])
                                w = wxy * wzs[dz]
                                if acc0 is None:
                                    acc0 = w * g0
                                    acc1 = w * g1
                                else:
                                    acc0 = acc0 + w * g0
                                    acc1 = acc1 + w * g1
                    fv[2 * l, sl] = acc0
                    fv[2 * l + 1, sl] = acc1
                return 0

            lax.fori_loop(0, _CNK // 16, pt_body, 0)
            pltpu.sync_copy(fv, ft_hbm.at[:, pl.ds(base, _CNK)])
            return 0

        lax.fori_loop(0, nchunk, chunk_body, 0)

    return enc(xs_a, ys_a, zs_a, grid_flat)


def _mlp_body(ft_ref, w1_ref, w2_ref, r1_ref, r2_ref, r3_ref, o_ref):
    f32 = jnp.float32
    cn = (((0,), (0,)), ((), ()))
    ft = ft_ref[...]
    t = lax.dot_general(w1_ref[...], ft, cn,
                        preferred_element_type=f32)
    t = jnp.maximum(t, 0.0)
    w21 = jnp.dot(w2_ref[...], r1_ref[...], preferred_element_type=f32)
    r = jnp.maximum(lax.dot_general(w21, t, cn,
                                    preferred_element_type=f32), 0.0)
    r = jnp.maximum(lax.dot_general(r2_ref[...], r, cn,
                                    preferred_element_type=f32), 0.0)
    o_ref[...] = jax.nn.sigmoid(
        lax.dot_general(r3_ref[...], r, cn, preferred_element_type=f32))


@functools.partial(jax.jit, static_argnames=("nb",))
def _run(x, grid, W1, W2, R1, R2, R3, nb=16384):
    n = x.shape[0]
    ft = _sc_encode(x[:, 0], x[:, 1], x[:, 2],
                    grid.transpose(0, 2, 1).reshape(-1))
    full = lambda a: pl.BlockSpec(a.shape, lambda i: (0,) * a.ndim)
    out = pl.pallas_call(
        _mlp_body,
        grid=(n // nb,),
        in_specs=[
            pl.BlockSpec((16, nb), lambda i: (0, i)),
            full(W1), full(W2), full(R1), full(R2), full(R3),
        ],
        out_specs=pl.BlockSpec((1, nb), lambda i: (0, i)),
        out_shape=jax.ShapeDtypeStruct((1, n), jnp.float32),
    )(ft, W1, W2, R1, R2, R3)
    return out.reshape(n, 1)


def kernel(x, grid, W1, W2, R1, R2, R3):
    return _run(x, grid, W1, W2, R1, R2, R3)


# speedup vs baseline: 1.1243x; 1.0342x over previous
import functools

import jax
import jax.numpy as jnp
import numpy as np
from jax import lax
from jax.experimental import pallas as pl
from jax.experimental.pallas import tpu as pltpu
from jax.experimental.pallas import tpu_sc as plsc

_L = 8
_T = 1024
_B_G = float(np.exp(np.log(20 * 0.5 / 2) / (_L - 1)))
_RES = [int(np.floor(2 * _B_G**l)) for l in range(_L)]
_C1 = np.int32(np.uint32(2654435761).astype(np.int32))
_C2 = np.int32(805459861)

_NW = 32
_CNK = 4096
_C1_2 = np.int32(np.uint32(2 * 2654435761 % (2**32)).astype(np.int32))
_C2_2 = np.int32(np.uint32(2 * 805459861 % (2**32)).astype(np.int32))


def _sc_encode(xs_a, ys_a, zs_a, grid_flat):
    n = xs_a.shape[0]
    npts = n // _NW
    nchunk = npts // _CNK
    mesh = plsc.VectorSubcoreMesh(core_axis_name="c", subcore_axis_name="s")

    @functools.partial(
        pl.kernel,
        out_type=jax.ShapeDtypeStruct((16, n), jnp.float32),
        mesh=mesh,
        scratch_types=[
            pltpu.VMEM((_L * _T * 2,), jnp.float32),
            pltpu.VMEM((_CNK,), jnp.float32),
            pltpu.VMEM((_CNK,), jnp.float32),
            pltpu.VMEM((_CNK,), jnp.float32),
            pltpu.VMEM((16, _CNK), jnp.float32),
        ],
        compiler_params=pltpu.CompilerParams(needs_layout_passes=False),
    )
    def enc(xs_hbm, ys_hbm, zs_hbm, grid_hbm, ft_hbm, gv, xv, yv, zv, fv):
        wid = lax.axis_index("s") * 2 + lax.axis_index("c")
        pltpu.sync_copy(grid_hbm, gv)
        base0 = wid * npts

        def chunk_body(ci, _):
            base = base0 + ci * _CNK
            pltpu.sync_copy(xs_hbm.at[pl.ds(base, _CNK)], xv)
            pltpu.sync_copy(ys_hbm.at[pl.ds(base, _CNK)], yv)
            pltpu.sync_copy(zs_hbm.at[pl.ds(base, _CNK)], zv)

            def pt_body(i, _):
                sl = pl.ds(i * 16, 16)
                xs = xv[sl]
                ys = yv[sl]
                zs = zv[sl]
                for l in range(_L):
                    res = float(_RES[l])
                    loff = l * _T * 2
                    px = xs * res
                    py = ys * res
                    pz = zs * res
                    ix = px.astype(jnp.int32)
                    iy = py.astype(jnp.int32)
                    iz = pz.astype(jnp.int32)
                    wx1 = px - ix.astype(jnp.float32)
                    wy1 = py - iy.astype(jnp.float32)
                    wz1 = pz - iz.astype(jnp.float32)
                    hy0 = iy * _C1
                    hz0 = iz * _C2
                    hx = (ix, ix + 1)
                    hy = (hy0, hy0 + _C1)
                    hz = (hz0, hz0 + _C2)
                    wxs = (1.0 - wx1, wx1)
                    wys = (1.0 - wy1, wy1)
                    wzs = (1.0 - wz1, wz1)
                    acc0 = None
                    acc1 = None
                    for dx in (0, 1):
                        for dy in (0, 1):
                            hxy = hx[dx] ^ hy[dy]
                            wxy = wxs[dx] * wys[dy]
                            for dz in (0, 1):
                                idx = ((hxy ^ hz[dz]) & (_T - 1)) | loff
                                g0 = plsc.load_gather(gv, [idx])
                                g1 = plsc.load_gather(gv, [idx + _T])
                                w = wxy * wzs[dz]
                                if acc0 is None:
                                    acc0 = w * g0
                                    acc1 = w * g1
                                else:
                                    acc0 = acc0 + w * g0
                                    acc1 = acc1 + w * g1
                    fv[2 * l, sl] = acc0
                    fv[2 * l + 1, sl] = acc1
                return 0

            lax.fori_loop(0, _CNK // 16, pt_body, 0)
            pltpu.sync_copy(fv, ft_hbm.at[:, pl.ds(base, _CNK)])
            return 0

        lax.fori_loop(0, nchunk, chunk_body, 0)

    return enc(xs_a, ys_a, zs_a, grid_flat)


def _mlp_body(ft_ref, w1_ref, w2_ref, r1_ref, r2_ref, r3_ref, o_ref):
    f32 = jnp.float32
    cn = (((0,), (0,)), ((), ()))
    ft = ft_ref[...]
    t = lax.dot_general(w1_ref[...], ft, cn,
                        preferred_element_type=f32)
    t = jnp.maximum(t, 0.0)
    w21 = jnp.dot(w2_ref[...], r1_ref[...], preferred_element_type=f32)
    r = jnp.maximum(lax.dot_general(w21, t, cn,
                                    preferred_element_type=f32), 0.0)
    r = jnp.maximum(lax.dot_general(r2_ref[...], r, cn,
                                    preferred_element_type=f32), 0.0)
    o_ref[...] = jax.nn.sigmoid(
        lax.dot_general(r3_ref[...], r, cn, preferred_element_type=f32))


def _mlp(ft, W1, W2, R1, R2, R3, nb=16384):
    n = ft.shape[1]
    full = lambda a: pl.BlockSpec(a.shape, lambda i: (0,) * a.ndim)
    return pl.pallas_call(
        _mlp_body,
        grid=(n // nb,),
        in_specs=[
            pl.BlockSpec((16, nb), lambda i: (0, i)),
            full(W1), full(W2), full(R1), full(R2), full(R3),
        ],
        out_specs=pl.BlockSpec((1, nb), lambda i: (0, i)),
        out_shape=jax.ShapeDtypeStruct((1, n), jnp.float32),
    )(ft, W1, W2, R1, R2, R3)


@jax.jit
def _run(x, grid, W1, W2, R1, R2, R3):
    n = x.shape[0]
    h = n // 2
    gf = grid.transpose(0, 2, 1).reshape(-1)
    ft1 = _sc_encode(x[:h, 0], x[:h, 1], x[:h, 2], gf)
    ft2 = _sc_encode(x[h:, 0], x[h:, 1], x[h:, 2], gf)
    o1 = _mlp(ft1, W1, W2, R1, R2, R3)
    o2 = _mlp(ft2, W1, W2, R1, R2, R3)
    return jnp.concatenate([o1, o2], axis=1).reshape(n, 1)


def kernel(x, grid, W1, W2, R1, R2, R3):
    return _run(x, grid, W1, W2, R1, R2, R3)


# speedup vs baseline: 2.1458x; 1.9086x over previous
import functools

import jax
import jax.numpy as jnp
import numpy as np
from jax import lax
from jax.experimental import pallas as pl
from jax.experimental.pallas import tpu as pltpu
from jax.experimental.pallas import tpu_sc as plsc

_L = 8
_T = 1024
_B_G = float(np.exp(np.log(20 * 0.5 / 2) / (_L - 1)))
_RES = [int(np.floor(2 * _B_G**l)) for l in range(_L)]
_C1 = np.int32(np.uint32(2654435761).astype(np.int32))
_C2 = np.int32(805459861)

_NW = 32
_CNK = 4096
_C1_2 = np.int32(np.uint32(2 * 2654435761 % (2**32)).astype(np.int32))
_C2_2 = np.int32(np.uint32(2 * 805459861 % (2**32)).astype(np.int32))


def _sc_encode(xs_a, ys_a, zs_a, grid_flat):
    n = xs_a.shape[0]
    npts = n // _NW
    nchunk = npts // _CNK
    mesh = plsc.VectorSubcoreMesh(core_axis_name="c", subcore_axis_name="s")

    @functools.partial(
        pl.kernel,
        out_type=jax.ShapeDtypeStruct((16, n), jnp.float32),
        mesh=mesh,
        scratch_types=[
            pltpu.VMEM((_L * _T * 2,), jnp.float32),
            pltpu.VMEM((_CNK,), jnp.float32),
            pltpu.VMEM((_CNK,), jnp.float32),
            pltpu.VMEM((_CNK,), jnp.float32),
            pltpu.VMEM((16, _CNK), jnp.float32),
        ],
        compiler_params=pltpu.CompilerParams(needs_layout_passes=False),
    )
    def enc(xs_hbm, ys_hbm, zs_hbm, grid_hbm, ft_hbm, gv, xv, yv, zv, fv):
        wid = lax.axis_index("s") * 2 + lax.axis_index("c")
        pltpu.sync_copy(grid_hbm, gv)
        base0 = wid * npts

        def chunk_body(ci, _):
            base = base0 + ci * _CNK
            pltpu.sync_copy(xs_hbm.at[pl.ds(base, _CNK)], xv)
            pltpu.sync_copy(ys_hbm.at[pl.ds(base, _CNK)], yv)
            pltpu.sync_copy(zs_hbm.at[pl.ds(base, _CNK)], zv)

            def pt_body(i, _):
                sl = pl.ds(i * 16, 16)
                xs = xv[sl]
                ys = yv[sl]
                zs = zv[sl]
                for l in range(_L):
                    res = float(_RES[l])
                    loff = l * _T * 2
                    px = xs * res
                    py = ys * res
                    pz = zs * res
                    ix = px.astype(jnp.int32)
                    iy = py.astype(jnp.int32)
                    iz = pz.astype(jnp.int32)
                    wx1 = px - ix.astype(jnp.float32)
                    wy1 = py - iy.astype(jnp.float32)
                    wz1 = pz - iz.astype(jnp.float32)
                    hy0 = iy * _C1
                    hz0 = iz * _C2
                    hx = (ix, ix + 1)
                    hy = (hy0, hy0 + _C1)
                    hz = (hz0, hz0 + _C2)
                    wxs = (1.0 - wx1, wx1)
                    wys = (1.0 - wy1, wy1)
                    wzs = (1.0 - wz1, wz1)
                    acc0 = None
                    acc1 = None
                    for dx in (0, 1):
                        for dy in (0, 1):
                            hxy = hx[dx] ^ hy[dy]
                            wxy = wxs[dx] * wys[dy]
                            for dz in (0, 1):
                                idx = ((hxy ^ hz[dz]) & (_T - 1)) | loff
                                g0 = plsc.load_gather(gv, [idx])
                                g1 = plsc.load_gather(gv, [idx + _T])
                                w = wxy * wzs[dz]
                                if acc0 is None:
                                    acc0 = w * g0
                                    acc1 = w * g1
                                else:
                                    acc0 = acc0 + w * g0
                                    acc1 = acc1 + w * g1
                    fv[2 * l, sl] = acc0
                    fv[2 * l + 1, sl] = acc1
                return 0

            lax.fori_loop(0, _CNK // 16, pt_body, 0)
            pltpu.sync_copy(fv, ft_hbm.at[:, pl.ds(base, _CNK)])
            return 0

        lax.fori_loop(0, nchunk, chunk_body, 0)

    return enc(xs_a, ys_a, zs_a, grid_flat)


def _mlp_body(ft_ref, w1_ref, w2_ref, r1_ref, r2_ref, r3_ref, o_ref):
    f32 = jnp.float32
    cn = (((0,), (0,)), ((), ()))
    ft = ft_ref[...]
    t = lax.dot_general(w1_ref[...], ft, cn,
                        preferred_element_type=f32)
    t = jnp.maximum(t, 0.0)
    w21 = jnp.dot(w2_ref[...], r1_ref[...], preferred_element_type=f32)
    r = jnp.maximum(lax.dot_general(w21, t, cn,
                                    preferred_element_type=f32), 0.0)
    r = jnp.maximum(lax.dot_general(r2_ref[...], r, cn,
                                    preferred_element_type=f32), 0.0)
    o_ref[...] = jax.nn.sigmoid(
        lax.dot_general(r3_ref[...], r, cn, preferred_element_type=f32))


def _mlp(ft, W1, W2, R1, R2, R3, nb=16384):
    n = ft.shape[1]
    full = lambda a: pl.BlockSpec(a.shape, lambda i: (0,) * a.ndim)
    return pl.pallas_call(
        _mlp_body,
        grid=(n // nb,),
        in_specs=[
            pl.BlockSpec((16, nb), lambda i: (0, i)),
            full(W1), full(W2), full(R1), full(R2), full(R3),
        ],
        out_specs=pl.BlockSpec((1, nb), lambda i: (0, i)),
        out_shape=jax.ShapeDtypeStruct((1, n), jnp.float32),
    )(ft, W1, W2, R1, R2, R3)


@jax.jit
def _run(x, grid, W1, W2, R1, R2, R3):
    n = x.shape[0]
    npiece = 4
    h = n // npiece
    gf = grid.transpose(0, 2, 1).reshape(-1)
    fts = [
        _sc_encode(x[i * h:(i + 1) * h, 0], x[i * h:(i + 1) * h, 1],
                   x[i * h:(i + 1) * h, 2], gf)
        for i in range(npiece)
    ]
    outs = [_mlp(ft, W1, W2, R1, R2, R3) for ft in fts]
    return jnp.concatenate(outs, axis=1).reshape(n, 1)


def kernel(x, grid, W1, W2, R1, R2, R3):
    return _run(x, grid, W1, W2, R1, R2, R3)
